# baseline (speedup 1.0000x reference)
import jax
import jax.numpy as jnp
from jax.experimental import pallas as pl
from jax.experimental.pallas import tpu as pltpu

D_MODEL = 4096
NUM_EXPERTS = 64
G = 2
S = 2048
CAP = 64
C_OUT = CAP - 1
BS = 256


def _router_body(x_ref, w_ref, b_ref, combine_ref, mask_ref, carry1, carry2):
    sb = pl.program_id(1)

    @pl.when(sb == 0)
    def _():
        carry1[...] = jnp.zeros_like(carry1)
        carry2[...] = jnp.zeros_like(carry2)

    x = x_ref[0]
    w = w_ref[...].astype(jnp.float32)
    logits = jnp.dot(x, w, preferred_element_type=jnp.float32)
    logits = logits + b_ref[0, 0, :].astype(jnp.float32)
    probs = jax.nn.softmax(logits, axis=-1)

    lane_e = jax.lax.broadcasted_iota(jnp.int32, (BS, NUM_EXPERTS), 1)
    m1 = jnp.max(probs, axis=-1, keepdims=True)
    i1 = jnp.min(jnp.where(probs == m1, lane_e, NUM_EXPERTS), axis=-1, keepdims=True)
    sel1 = lane_e == i1
    pex = jnp.where(sel1, -1.0, probs)
    m2 = jnp.max(pex, axis=-1, keepdims=True)
    i2 = jnp.min(jnp.where(pex == m2, lane_e, NUM_EXPERTS), axis=-1, keepdims=True)
    sel2 = lane_e == i2

    mh1 = sel1.astype(jnp.float32)
    mh2 = sel2.astype(jnp.float32)
    r = jax.lax.broadcasted_iota(jnp.int32, (BS, BS), 0)
    c = jax.lax.broadcasted_iota(jnp.int32, (BS, BS), 1)
    tril = (r >= c).astype(jnp.float32)
    cum1 = jnp.dot(tril, mh1, preferred_element_type=jnp.float32) + carry1[...]
    cum2 = jnp.dot(tril, mh2, preferred_element_type=jnp.float32) + carry2[...]
    carry1[...] += jnp.sum(mh1, axis=0, keepdims=True)
    carry2[...] += jnp.sum(mh2, axis=0, keepdims=True)

    g2d = m1 * mh1 + m2 * mh2
    p2d = cum1 * mh1 + cum2 * mh2 - 1.0

    lane_c3 = jax.lax.broadcasted_iota(jnp.int32, (BS, NUM_EXPERTS, C_OUT), 2)
    p2i = p2d.astype(jnp.int32)
    hit = lane_c3 == p2i[:, :, None]
    out = jnp.where(hit, g2d[:, :, None], 0.0)
    combine_ref[0] = out
    mask_ref[0] = (out != 0.0).astype(jnp.int8)


def kernel(x, gate_weight, gate_bias, expert_capacity):
    del expert_capacity
    grid = (G, S // BS)
    combine, mask = pl.pallas_call(
        _router_body,
        grid=grid,
        in_specs=[
            pl.BlockSpec((1, BS, D_MODEL), lambda g, s: (g, s, 0)),
            pl.BlockSpec((D_MODEL, NUM_EXPERTS), lambda g, s: (0, 0)),
            pl.BlockSpec((1, 1, NUM_EXPERTS), lambda g, s: (0, 0, 0)),
        ],
        out_specs=[
            pl.BlockSpec((1, BS, NUM_EXPERTS, C_OUT), lambda g, s: (g, s, 0, 0)),
            pl.BlockSpec((1, BS, NUM_EXPERTS, C_OUT), lambda g, s: (g, s, 0, 0)),
        ],
        out_shape=[
            jax.ShapeDtypeStruct((G, S, NUM_EXPERTS, C_OUT), jnp.float32),
            jax.ShapeDtypeStruct((G, S, NUM_EXPERTS, C_OUT), jnp.int8),
        ],
        scratch_shapes=[
            pltpu.VMEM((1, NUM_EXPERTS), jnp.float32),
            pltpu.VMEM((1, NUM_EXPERTS), jnp.float32),
        ],
    )(x, gate_weight, gate_bias)
    return combine, mask.astype(jnp.bool_)
